# baseline (speedup 1.0000x reference)
import functools

import jax
import jax.numpy as jnp
from jax.experimental import pallas as pl
from jax.experimental.pallas import tpu as pltpu

CODEBOOK_SIZE = 512
DIM = 32
BETA = 0.25
_G = 8
_TT = 1024


def _vq_kernel(x_ref, cb_ref, cbt_ref, quant_ref, idx_ref, loss_ref, m_ref):
    cb = cb_ref[...]
    tlen = x_ref.shape[2]
    cbn2 = cb * jnp.float32(-2.0)
    c2 = jnp.sum(cb * cb, axis=1)[:, None]
    cbt = cbt_ref[...]
    loss_part = jnp.zeros((1, 1), jnp.float32)

    for t0 in range(0, tlen, _TT):
        xb = x_ref[0, :, t0:t0 + _TT]
        m_ref[:, t0:t0 + _TT] = jax.lax.dot_general(
            cbn2, xb, (((1,), (0,)), ((), ())),
            precision=jax.lax.Precision.DEFAULT,
            preferred_element_type=jnp.float32,
        )
        a = jnp.sum(xb * xb, axis=0, keepdims=True)

        run_min = jnp.full((_G, _TT), jnp.inf, jnp.float32)
        run_g = jnp.zeros((_G, _TT), jnp.int32)
        for g in range(CODEBOOK_SIZE // _G):
            mg = m_ref[g * _G:(g + 1) * _G, t0:t0 + _TT]
            d = (a + mg) + c2[g * _G:(g + 1) * _G]
            better = d < run_min
            run_min = jnp.minimum(run_min, d)
            run_g = jnp.where(better, g, run_g)
        siota = jax.lax.broadcasted_iota(jnp.int32, (_G, _TT), 0)
        run_j = run_g * _G + siota
        gmin = jnp.min(run_min, axis=0, keepdims=True)
        idx = jnp.min(jnp.where(run_min == gmin, run_j, CODEBOOK_SIZE),
                      axis=0)
        idx_ref[0, 0, t0:t0 + _TT] = idx

        idx_lo = jnp.broadcast_to((idx & 127)[None, :], (DIM, _TT))
        chunk = jnp.broadcast_to((idx >> 7)[None, :], (DIM, _TT))
        quant = jnp.zeros((DIM, _TT), jnp.float32)
        for c in range(CODEBOOK_SIZE // 128):
            part = jnp.take_along_axis(
                cbt[:, c * 128:(c + 1) * 128], idx_lo,
                axis=1, mode="promise_in_bounds",
            )
            quant = jnp.where(chunk == c, part, quant)
        diff = quant - xb
        quant_ref[0, :, t0:t0 + _TT] = xb + diff
        loss_part += jnp.sum(diff * diff).reshape(1, 1)

    @pl.when(pl.program_id(0) == 0)
    def _():
        loss_ref[...] = jnp.zeros_like(loss_ref)
    loss_ref[...] += loss_part


@functools.partial(jax.jit, static_argnames=())
def kernel(x, codebook):
    bsz, dim, tlen = x.shape
    quant, idx3, loss_sum = pl.pallas_call(
        _vq_kernel,
        grid=(bsz,),
        in_specs=[
            pl.BlockSpec((1, dim, tlen), lambda b: (b, 0, 0)),
            pl.BlockSpec((CODEBOOK_SIZE, dim), lambda b: (0, 0)),
            pl.BlockSpec((dim, CODEBOOK_SIZE), lambda b: (0, 0)),
        ],
        out_specs=[
            pl.BlockSpec((1, dim, tlen), lambda b: (b, 0, 0)),
            pl.BlockSpec((1, 1, tlen), lambda b: (b, 0, 0)),
            pl.BlockSpec((1, 1), lambda b: (0, 0)),
        ],
        out_shape=[
            jax.ShapeDtypeStruct((bsz, dim, tlen), jnp.float32),
            jax.ShapeDtypeStruct((bsz, 1, tlen), jnp.int32),
            jax.ShapeDtypeStruct((1, 1), jnp.float32),
        ],
        scratch_shapes=[pltpu.VMEM((CODEBOOK_SIZE, tlen), jnp.float32)],
    )(x, codebook, codebook.T)
    idx = idx3.reshape(bsz, tlen)
    loss_vq = loss_sum[0, 0] / jnp.float32(bsz * dim * tlen)
    loss_commit = jnp.float32(BETA) * loss_vq
    return (quant, idx, loss_vq, loss_commit)


# speedup vs baseline: 1.1262x; 1.1262x over previous
import functools

import jax
import jax.numpy as jnp
from jax.experimental import pallas as pl
from jax.experimental.pallas import tpu as pltpu

CODEBOOK_SIZE = 512
DIM = 32
BETA = 0.25
_G = 8


def _vq_kernel(x_ref, cb_ref, cbt_ref, quant_ref, idx_ref, loss_ref, m_ref):
    xb = x_ref[0]
    cb = cb_ref[...]
    tlen = xb.shape[1]
    cbn2 = cb * jnp.float32(-2.0)
    a = jnp.sum(xb * xb, axis=0, keepdims=True)
    c2 = jnp.sum(cb * cb, axis=1)[:, None]

    m_ref[...] = jax.lax.dot_general(
        cbn2, xb, (((1,), (0,)), ((), ())),
        precision=jax.lax.Precision.DEFAULT,
        preferred_element_type=jnp.float32,
    )

    run_min = jnp.full((_G, tlen), jnp.inf, jnp.float32)
    run_g = jnp.zeros((_G, tlen), jnp.int32)
    for g in range(CODEBOOK_SIZE // _G):
        mg = m_ref[g * _G:(g + 1) * _G, :]
        d = (a + mg) + c2[g * _G:(g + 1) * _G]
        better = d < run_min
        run_min = jnp.minimum(run_min, d)
        run_g = jnp.where(better, g, run_g)
    siota = jax.lax.broadcasted_iota(jnp.int32, (_G, tlen), 0)
    run_j = run_g * _G + siota
    gmin = jnp.min(run_min, axis=0, keepdims=True)
    idx = jnp.min(jnp.where(run_min == gmin, run_j, CODEBOOK_SIZE),
                  axis=0)
    idx_ref[0, 0, :] = idx

    idx_lo = jnp.broadcast_to((idx & 127)[None, :], (DIM, tlen))
    chunk = jnp.broadcast_to((idx >> 7)[None, :], (DIM, tlen))
    quant = jnp.zeros((DIM, tlen), jnp.float32)
    for c in range(CODEBOOK_SIZE // 128):
        part = jnp.take_along_axis(
            cbt_ref[:, c * 128:(c + 1) * 128], idx_lo,
            axis=1, mode="promise_in_bounds",
        )
        quant = jnp.where(chunk == c, part, quant)
    diff = quant - xb
    quant_ref[0] = xb + diff
    part = jnp.sum(diff * diff).reshape(1, 1)
    @pl.when(pl.program_id(0) == 0)
    def _():
        loss_ref[...] = jnp.zeros_like(loss_ref)
    loss_ref[...] += part


@functools.partial(jax.jit, static_argnames=())
def kernel(x, codebook):
    bsz, dim, tlen = x.shape
    quant, idx3, loss_sum = pl.pallas_call(
        _vq_kernel,
        grid=(bsz,),
        in_specs=[
            pl.BlockSpec((1, dim, tlen), lambda b: (b, 0, 0)),
            pl.BlockSpec((CODEBOOK_SIZE, dim), lambda b: (0, 0)),
            pl.BlockSpec((dim, CODEBOOK_SIZE), lambda b: (0, 0)),
        ],
        out_specs=[
            pl.BlockSpec((1, dim, tlen), lambda b: (b, 0, 0)),
            pl.BlockSpec((1, 1, tlen), lambda b: (b, 0, 0)),
            pl.BlockSpec((1, 1), lambda b: (0, 0)),
        ],
        out_shape=[
            jax.ShapeDtypeStruct((bsz, dim, tlen), jnp.float32),
            jax.ShapeDtypeStruct((bsz, 1, tlen), jnp.int32),
            jax.ShapeDtypeStruct((1, 1), jnp.float32),
        ],
        scratch_shapes=[pltpu.VMEM((CODEBOOK_SIZE, tlen), jnp.float32)],
    )(x, codebook, codebook.T)
    idx = idx3.reshape(bsz, tlen)
    loss_vq = loss_sum[0, 0] / jnp.float32(bsz * dim * tlen)
    loss_commit = jnp.float32(BETA) * loss_vq
    return (quant, idx, loss_vq, loss_commit)


# speedup vs baseline: 1.2749x; 1.1320x over previous
import functools

import jax
import jax.numpy as jnp
from jax.experimental import pallas as pl
from jax.experimental.pallas import tpu as pltpu

CODEBOOK_SIZE = 512
DIM = 32
BETA = 0.25
_G = 8


def _vq_kernel(x_ref, cb_ref, cbt_ref, quant_ref, idx_ref, loss_ref,
               m_ref0, m_ref1):
    cb = cb_ref[...]
    tlen = x_ref.shape[2]
    cbn2 = cb * jnp.float32(-2.0)
    c2 = jnp.sum(cb * cb, axis=1)[:, None]
    loss_part = jnp.zeros((1, 1), jnp.float32)

    for i, m_ref in enumerate((m_ref0, m_ref1)):
        xb = x_ref[i]
        a = jnp.sum(xb * xb, axis=0, keepdims=True)
        m_ref[...] = jax.lax.dot_general(
            cbn2, xb, (((1,), (0,)), ((), ())),
            precision=jax.lax.Precision.DEFAULT,
            preferred_element_type=jnp.float32,
        )

        run_min = jnp.full((_G, tlen), jnp.inf, jnp.float32)
        run_g = jnp.zeros((_G, tlen), jnp.int32)
        for g in range(CODEBOOK_SIZE // _G):
            mg = m_ref[g * _G:(g + 1) * _G, :]
            d = (a + mg) + c2[g * _G:(g + 1) * _G]
            better = d < run_min
            run_min = jnp.minimum(run_min, d)
            run_g = jnp.where(better, g, run_g)
        siota = jax.lax.broadcasted_iota(jnp.int32, (_G, tlen), 0)
        run_j = run_g * _G + siota
        gmin = jnp.min(run_min, axis=0, keepdims=True)
        idx = jnp.min(jnp.where(run_min == gmin, run_j, CODEBOOK_SIZE),
                      axis=0)
        idx_ref[i, 0, :] = idx

        idx_lo = jnp.broadcast_to((idx & 127)[None, :], (DIM, tlen))
        chunk = jnp.broadcast_to((idx >> 7)[None, :], (DIM, tlen))
        quant = jnp.zeros((DIM, tlen), jnp.float32)
        for c in range(CODEBOOK_SIZE // 128):
            part = jnp.take_along_axis(
                cbt_ref[:, c * 128:(c + 1) * 128], idx_lo,
                axis=1, mode="promise_in_bounds",
            )
            quant = jnp.where(chunk == c, part, quant)
        diff = quant - xb
        quant_ref[i] = xb + diff
        loss_part += jnp.sum(diff * diff).reshape(1, 1)

    @pl.when(pl.program_id(0) == 0)
    def _():
        loss_ref[...] = jnp.zeros_like(loss_ref)
    loss_ref[...] += loss_part


@functools.partial(jax.jit, static_argnames=())
def kernel(x, codebook):
    bsz, dim, tlen = x.shape
    quant, idx3, loss_sum = pl.pallas_call(
        _vq_kernel,
        grid=(bsz // 2,),
        in_specs=[
            pl.BlockSpec((2, dim, tlen), lambda b: (b, 0, 0)),
            pl.BlockSpec((CODEBOOK_SIZE, dim), lambda b: (0, 0)),
            pl.BlockSpec((dim, CODEBOOK_SIZE), lambda b: (0, 0)),
        ],
        out_specs=[
            pl.BlockSpec((2, dim, tlen), lambda b: (b, 0, 0)),
            pl.BlockSpec((2, 1, tlen), lambda b: (b, 0, 0)),
            pl.BlockSpec((1, 1), lambda b: (0, 0)),
        ],
        out_shape=[
            jax.ShapeDtypeStruct((bsz, dim, tlen), jnp.float32),
            jax.ShapeDtypeStruct((bsz, 1, tlen), jnp.int32),
            jax.ShapeDtypeStruct((1, 1), jnp.float32),
        ],
        scratch_shapes=[pltpu.VMEM((CODEBOOK_SIZE, tlen), jnp.float32),
                        pltpu.VMEM((CODEBOOK_SIZE, tlen), jnp.float32)],
    )(x, codebook, codebook.T)
    idx = idx3.reshape(bsz, tlen)
    loss_vq = loss_sum[0, 0] / jnp.float32(bsz * dim * tlen)
    loss_commit = jnp.float32(BETA) * loss_vq
    return (quant, idx, loss_vq, loss_commit)


# speedup vs baseline: 1.3114x; 1.0286x over previous
import functools

import jax
import jax.numpy as jnp
from jax.experimental import pallas as pl
from jax.experimental.pallas import tpu as pltpu

CODEBOOK_SIZE = 512
DIM = 32
BETA = 0.25
_G = 8


def _vq_kernel(x_ref, cb_ref, cbt_ref, quant_ref, idx_ref, loss_ref,
               m_ref0, m_ref1, m_ref2, m_ref3):
    cb = cb_ref[...]
    tlen = x_ref.shape[2]
    cbn2 = cb * jnp.float32(-2.0)
    c2 = jnp.sum(cb * cb, axis=1)[:, None]
    loss_part = jnp.zeros((1, 1), jnp.float32)

    for i, m_ref in enumerate((m_ref0, m_ref1, m_ref2, m_ref3)):
        xb = x_ref[i]
        a = jnp.sum(xb * xb, axis=0, keepdims=True)
        m_ref[...] = jax.lax.dot_general(
            cbn2, xb, (((1,), (0,)), ((), ())),
            precision=jax.lax.Precision.DEFAULT,
            preferred_element_type=jnp.float32,
        )

        run_min = jnp.full((_G, tlen), jnp.inf, jnp.float32)
        run_g = jnp.zeros((_G, tlen), jnp.int32)
        for g in range(CODEBOOK_SIZE // _G):
            mg = m_ref[g * _G:(g + 1) * _G, :]
            d = (a + mg) + c2[g * _G:(g + 1) * _G]
            better = d < run_min
            run_min = jnp.minimum(run_min, d)
            run_g = jnp.where(better, g, run_g)
        siota = jax.lax.broadcasted_iota(jnp.int32, (_G, tlen), 0)
        run_j = run_g * _G + siota
        gmin = jnp.min(run_min, axis=0, keepdims=True)
        idx = jnp.min(jnp.where(run_min == gmin, run_j, CODEBOOK_SIZE),
                      axis=0)
        idx_ref[i, 0, :] = idx

        idx_lo = jnp.broadcast_to((idx & 127)[None, :], (DIM, tlen))
        chunk = jnp.broadcast_to((idx >> 7)[None, :], (DIM, tlen))
        quant = jnp.zeros((DIM, tlen), jnp.float32)
        for c in range(CODEBOOK_SIZE // 128):
            part = jnp.take_along_axis(
                cbt_ref[:, c * 128:(c + 1) * 128], idx_lo,
                axis=1, mode="promise_in_bounds",
            )
            quant = jnp.where(chunk == c, part, quant)
        diff = quant - xb
        quant_ref[i] = xb + diff
        loss_part += jnp.sum(diff * diff).reshape(1, 1)

    @pl.when(pl.program_id(0) == 0)
    def _():
        loss_ref[...] = jnp.zeros_like(loss_ref)
    loss_ref[...] += loss_part


@functools.partial(jax.jit, static_argnames=())
def kernel(x, codebook):
    bsz, dim, tlen = x.shape
    quant, idx3, loss_sum = pl.pallas_call(
        _vq_kernel,
        grid=(bsz // 4,),
        in_specs=[
            pl.BlockSpec((4, dim, tlen), lambda b: (b, 0, 0)),
            pl.BlockSpec((CODEBOOK_SIZE, dim), lambda b: (0, 0)),
            pl.BlockSpec((dim, CODEBOOK_SIZE), lambda b: (0, 0)),
        ],
        out_specs=[
            pl.BlockSpec((4, dim, tlen), lambda b: (b, 0, 0)),
            pl.BlockSpec((4, 1, tlen), lambda b: (b, 0, 0)),
            pl.BlockSpec((1, 1), lambda b: (0, 0)),
        ],
        out_shape=[
            jax.ShapeDtypeStruct((bsz, dim, tlen), jnp.float32),
            jax.ShapeDtypeStruct((bsz, 1, tlen), jnp.int32),
            jax.ShapeDtypeStruct((1, 1), jnp.float32),
        ],
        scratch_shapes=[pltpu.VMEM((CODEBOOK_SIZE, tlen), jnp.float32)
                        for _ in range(4)],
    )(x, codebook, codebook.T)
    idx = idx3.reshape(bsz, tlen)
    loss_vq = loss_sum[0, 0] / jnp.float32(bsz * dim * tlen)
    loss_commit = jnp.float32(BETA) * loss_vq
    return (quant, idx, loss_vq, loss_commit)


# speedup vs baseline: 1.3218x; 1.0080x over previous
import functools

import jax
import jax.numpy as jnp
from jax.experimental import pallas as pl
from jax.experimental.pallas import tpu as pltpu

CODEBOOK_SIZE = 512
DIM = 32
BETA = 0.25
_G = 8


def _vq_kernel(x_ref, cb_ref, cbt_ref, quant_ref, idx_ref, loss_ref,
               *m_refs):
    cb = cb_ref[...]
    tlen = x_ref.shape[2]
    cbn2 = cb * jnp.float32(-2.0)
    c2 = jnp.sum(cb * cb, axis=1)[:, None]
    loss_part = jnp.zeros((1, 1), jnp.float32)

    for i, m_ref in enumerate(m_refs):
        xb = x_ref[i]
        a = jnp.sum(xb * xb, axis=0, keepdims=True)
        m_ref[...] = jax.lax.dot_general(
            cbn2, xb, (((1,), (0,)), ((), ())),
            precision=jax.lax.Precision.DEFAULT,
            preferred_element_type=jnp.float32,
        )

        run_min = jnp.full((_G, tlen), jnp.inf, jnp.float32)
        run_g = jnp.zeros((_G, tlen), jnp.int32)
        for g in range(CODEBOOK_SIZE // _G):
            mg = m_ref[g * _G:(g + 1) * _G, :]
            d = (a + mg) + c2[g * _G:(g + 1) * _G]
            better = d < run_min
            run_min = jnp.minimum(run_min, d)
            run_g = jnp.where(better, g, run_g)
        siota = jax.lax.broadcasted_iota(jnp.int32, (_G, tlen), 0)
        run_j = run_g * _G + siota
        gmin = jnp.min(run_min, axis=0, keepdims=True)
        idx = jnp.min(jnp.where(run_min == gmin, run_j, CODEBOOK_SIZE),
                      axis=0)
        idx_ref[i, 0, :] = idx

        idx_lo = jnp.broadcast_to((idx & 127)[None, :], (DIM, tlen))
        chunk = jnp.broadcast_to((idx >> 7)[None, :], (DIM, tlen))
        quant = jnp.zeros((DIM, tlen), jnp.float32)
        for c in range(CODEBOOK_SIZE // 128):
            part = jnp.take_along_axis(
                cbt_ref[:, c * 128:(c + 1) * 128], idx_lo,
                axis=1, mode="promise_in_bounds",
            )
            quant = jnp.where(chunk == c, part, quant)
        diff = quant - xb
        quant_ref[i] = xb + diff
        loss_part += jnp.sum(diff * diff).reshape(1, 1)

    @pl.when(pl.program_id(0) == 0)
    def _():
        loss_ref[...] = jnp.zeros_like(loss_ref)
    loss_ref[...] += loss_part


@functools.partial(jax.jit, static_argnames=())
def kernel(x, codebook):
    bsz, dim, tlen = x.shape
    quant, idx3, loss_sum = pl.pallas_call(
        _vq_kernel,
        grid=(bsz // 8,),
        in_specs=[
            pl.BlockSpec((8, dim, tlen), lambda b: (b, 0, 0)),
            pl.BlockSpec((CODEBOOK_SIZE, dim), lambda b: (0, 0)),
            pl.BlockSpec((dim, CODEBOOK_SIZE), lambda b: (0, 0)),
        ],
        out_specs=[
            pl.BlockSpec((8, dim, tlen), lambda b: (b, 0, 0)),
            pl.BlockSpec((8, 1, tlen), lambda b: (b, 0, 0)),
            pl.BlockSpec((1, 1), lambda b: (0, 0)),
        ],
        out_shape=[
            jax.ShapeDtypeStruct((bsz, dim, tlen), jnp.float32),
            jax.ShapeDtypeStruct((bsz, 1, tlen), jnp.int32),
            jax.ShapeDtypeStruct((1, 1), jnp.float32),
        ],
        scratch_shapes=[pltpu.VMEM((CODEBOOK_SIZE, tlen), jnp.float32)
                        for _ in range(8)],
    )(x, codebook, codebook.T)
    idx = idx3.reshape(bsz, tlen)
    loss_vq = loss_sum[0, 0] / jnp.float32(bsz * dim * tlen)
    loss_commit = jnp.float32(BETA) * loss_vq
    return (quant, idx, loss_vq, loss_commit)
